# Initial kernel scaffold; baseline (speedup 1.0000x reference)
import jax
import jax.numpy as jnp
from jax.experimental import pallas as pl


def kernel(features, w1, w2, dw1, db1, dw2, db2, dw3, db3, dw4, db4, dw5, db5, src_nodes0, dstsrc2src0_1, dstsrc2src0_2, dstsrc2dst0_1, dstsrc2dst0_2, dif_mat0_1, dif_mat0_2, src_nodes1, dstsrc2src1_1, dstsrc2src1_2, dstsrc2dst1_1, dstsrc2dst1_2, dif_mat1_1, dif_mat1_2):
    raise NotImplementedError("write your pallas kernel here")


# speedup vs baseline: 2.2188x; 2.2188x over previous
import functools

import jax
import jax.numpy as jnp
from jax import lax
from jax.experimental import pallas as pl
from jax.experimental.pallas import tpu as pltpu
from jax.experimental.pallas import tpu_sc as plsc

N_NODES = 100000
D = 128
B = 256
N1 = 2048
N2 = 8192

NC = 2
NS = 16
NW = NC * NS
L = 16

CHUNK = 128


def _compose_and_gather(wid, feat, nodes_v, sel_hbm, out_hbm, sel_v, idx_v,
                        rows_v, sem, count):
    base = wid * count
    pltpu.sync_copy(sel_hbm.at[pl.ds(base, count)], sel_v.at[pl.ds(0, count)])
    for o in range(0, count, L):
        sel16 = sel_v[pl.ds(o, L)]
        idx_v[pl.ds(o, L)] = plsc.load_gather(nodes_v, [sel16])
    for c in range(0, count, CHUNK):
        n = min(CHUNK, count - c)
        pltpu.async_copy(feat.at[idx_v.at[pl.ds(c, n)]],
                         rows_v.at[pl.ds(c, n)], sem).wait()
    pltpu.sync_copy(rows_v.at[pl.ds(0, count)], out_hbm.at[pl.ds(base, count)])


def _direct_gather(wid, table, idx_hbm, out_hbm, idx_v, rows_v, sem, count):
    base = wid * count
    pltpu.sync_copy(idx_hbm.at[pl.ds(base, count)], idx_v.at[pl.ds(0, count)])
    for c in range(0, count, CHUNK):
        n = min(CHUNK, count - c)
        pltpu.async_copy(table.at[idx_v.at[pl.ds(c, n)]],
                         rows_v.at[pl.ds(c, n)], sem).wait()
    pltpu.sync_copy(rows_v.at[pl.ds(0, count)], out_hbm.at[pl.ds(base, count)])


def _sc_feature_gather(features, nodes0, s2_0, d2_0, nodes1, s2_1, d2_1):
    mesh = plsc.VectorSubcoreMesh(core_axis_name="c", subcore_axis_name="s")
    fdt = jax.ShapeDtypeStruct

    def body(feat, n0, sA0, dA0, n1, sA1, dA1,
             src0, dst0, src1, dst1,
             nodes_v0, nodes_v1, sel_v, idx_v, rows_v, sem):
        wid = lax.axis_index("s") * NC + lax.axis_index("c")
        pltpu.sync_copy(n0, nodes_v0)
        pltpu.sync_copy(n1, nodes_v1)
        _compose_and_gather(wid, feat, nodes_v0, sA0, src0, sel_v, idx_v,
                            rows_v, sem, N2 // NW)
        _compose_and_gather(wid, feat, nodes_v0, dA0, dst0, sel_v, idx_v,
                            rows_v, sem, N1 // NW)
        _compose_and_gather(wid, feat, nodes_v1, sA1, src1, sel_v, idx_v,
                            rows_v, sem, N2 // NW)
        _compose_and_gather(wid, feat, nodes_v1, dA1, dst1, sel_v, idx_v,
                            rows_v, sem, N1 // NW)

    return pl.kernel(
        body,
        out_type=(fdt((N2, D), jnp.float32), fdt((N1, D), jnp.float32),
                  fdt((N2, D), jnp.float32), fdt((N1, D), jnp.float32)),
        mesh=mesh,
        scratch_types=(
            pltpu.VMEM((N2,), jnp.int32),
            pltpu.VMEM((N2,), jnp.int32),
            pltpu.VMEM((N2 // NW,), jnp.int32),
            pltpu.VMEM((N2 // NW,), jnp.int32),
            pltpu.VMEM((N2 // NW, D), jnp.float32),
            pltpu.SemaphoreType.DMA,
        ),
        compiler_params=pltpu.CompilerParams(needs_layout_passes=False),
    )(features, nodes0, s2_0, d2_0, nodes1, s2_1, d2_1)


def _sc_h2_gather(h2_0, h2_1, s1_0, d1_0, s1_1, d1_1):
    mesh = plsc.VectorSubcoreMesh(core_axis_name="c", subcore_axis_name="s")
    fdt = jax.ShapeDtypeStruct

    def body(t0, t1, sA0, dA0, sA1, dA1,
             src0, dst0, src1, dst1,
             idx_v, rows_v, sem):
        wid = lax.axis_index("s") * NC + lax.axis_index("c")
        _direct_gather(wid, t0, sA0, src0, idx_v, rows_v, sem, N1 // NW)
        _direct_gather(wid, t0, dA0, dst0, idx_v, rows_v, sem, B // NW)
        _direct_gather(wid, t1, sA1, src1, idx_v, rows_v, sem, N1 // NW)
        _direct_gather(wid, t1, dA1, dst1, idx_v, rows_v, sem, B // NW)

    return pl.kernel(
        body,
        out_type=(fdt((N1, D), jnp.float32), fdt((B, D), jnp.float32),
                  fdt((N1, D), jnp.float32), fdt((B, D), jnp.float32)),
        mesh=mesh,
        scratch_types=(
            pltpu.VMEM((N1 // NW,), jnp.int32),
            pltpu.VMEM((N1 // NW, D), jnp.float32),
            pltpu.SemaphoreType.DMA,
        ),
        compiler_params=pltpu.CompilerParams(needs_layout_passes=False),
    )(h2_0, h2_1, s1_0, d1_0, s1_1, d1_1)


KBLK = 512
KB = N2 // KBLK


def _tc_layer2_body(dm0_ref, dm1_ref, src0_ref, src1_ref, dst0_ref, dst1_ref,
                    w1_ref, h20_ref, h21_ref, acc0, acc1):
    k = pl.program_id(0)

    @pl.when(k == 0)
    def _():
        acc0[...] = jnp.zeros_like(acc0)
        acc1[...] = jnp.zeros_like(acc1)

    acc0[...] += jnp.dot(dm0_ref[...], src0_ref[...],
                         preferred_element_type=jnp.float32)
    acc1[...] += jnp.dot(dm1_ref[...], src1_ref[...],
                         preferred_element_type=jnp.float32)

    @pl.when(k == KB - 1)
    def _():
        w1t = w1_ref[0:D, :]
        w1b = w1_ref[D:2 * D, :]
        h20_ref[...] = jnp.maximum(
            jnp.dot(acc0[...], w1t, preferred_element_type=jnp.float32)
            + jnp.dot(dst0_ref[...], w1b, preferred_element_type=jnp.float32),
            0.0)
        h21_ref[...] = jnp.maximum(
            jnp.dot(acc1[...], w1t, preferred_element_type=jnp.float32)
            + jnp.dot(dst1_ref[...], w1b, preferred_element_type=jnp.float32),
            0.0)


def _tc_layer2(dm0, dm1, src0, src1, dst0, dst1, w1):
    fdt = jax.ShapeDtypeStruct
    return pl.pallas_call(
        _tc_layer2_body,
        grid=(KB,),
        in_specs=[
            pl.BlockSpec((N1, KBLK), lambda k: (0, k)),
            pl.BlockSpec((N1, KBLK), lambda k: (0, k)),
            pl.BlockSpec((KBLK, D), lambda k: (k, 0)),
            pl.BlockSpec((KBLK, D), lambda k: (k, 0)),
            pl.BlockSpec((N1, D), lambda k: (0, 0)),
            pl.BlockSpec((N1, D), lambda k: (0, 0)),
            pl.BlockSpec((2 * D, D), lambda k: (0, 0)),
        ],
        out_specs=[
            pl.BlockSpec((N1, D), lambda k: (0, 0)),
            pl.BlockSpec((N1, D), lambda k: (0, 0)),
        ],
        out_shape=[fdt((N1, D), jnp.float32), fdt((N1, D), jnp.float32)],
        scratch_shapes=[pltpu.VMEM((N1, D), jnp.float32),
                        pltpu.VMEM((N1, D), jnp.float32)],
    )(dm0, dm1, src0, src1, dst0, dst1, w1)


def _tc_head_body(s10_ref, d10_ref, s11_ref, d11_ref, dmA_ref, dmB_ref,
                  w2_ref, dw1_ref, db1_ref, dw2_ref, db2_ref, dw3_ref,
                  db3_ref, dw4_ref, db4_ref, w5r_ref, db5_ref, out_ref):
    f32 = jnp.float32
    w2t = w2_ref[0:D, :]
    w2b = w2_ref[D:2 * D, :]

    agg0 = jnp.dot(dmA_ref[...], s10_ref[...], preferred_element_type=f32)
    h10 = (jnp.dot(agg0, w2t, preferred_element_type=f32)
           + jnp.dot(d10_ref[...], w2b, preferred_element_type=f32))
    agg1 = jnp.dot(dmB_ref[...], s11_ref[...], preferred_element_type=f32)
    h11 = (jnp.dot(agg1, w2t, preferred_element_type=f32)
           + jnp.dot(d11_ref[...], w2b, preferred_element_type=f32))

    ss = jnp.sum(h10 * h10 + h11 * h11, axis=1, keepdims=True)
    scale = lax.rsqrt(jnp.maximum(ss, 1e-12))

    dw1t = dw1_ref[0:D, :]
    dw1b = dw1_ref[D:2 * D, :]
    h = (jnp.dot(h10, dw1t, preferred_element_type=f32)
         + jnp.dot(h11, dw1b, preferred_element_type=f32)) * scale
    h = jnp.maximum(h + db1_ref[...], 0.0)
    h = jnp.maximum(jnp.dot(h, dw2_ref[...], preferred_element_type=f32)
                    + db2_ref[...], 0.0)
    h = jnp.maximum(jnp.dot(h, dw3_ref[...], preferred_element_type=f32)
                    + db3_ref[...], 0.0)
    h = jnp.maximum(jnp.dot(h, dw4_ref[...], preferred_element_type=f32)
                    + db4_ref[...], 0.0)
    z = jnp.sum(h * w5r_ref[...], axis=1, keepdims=True) + db5_ref[...]
    m = jnp.max(z, axis=1, keepdims=True)
    e = jnp.exp(z - m)
    out_ref[...] = e / jnp.sum(e, axis=1, keepdims=True)


def _tc_head(s10, d10, s11, d11, dmA, dmB, w2,
             dw1, db1, dw2, db2, dw3, db3, dw4, db4, dw5, db5):
    fdt = jax.ShapeDtypeStruct
    args = (s10, d10, s11, d11, dmA, dmB, w2,
            dw1, db1.reshape(1, -1), dw2, db2.reshape(1, -1),
            dw3, db3.reshape(1, -1), dw4, db4.reshape(1, -1),
            dw5.reshape(1, -1), db5.reshape(1, -1))
    return pl.pallas_call(
        _tc_head_body,
        out_shape=fdt((B, 1), jnp.float32),
    )(*args)


def kernel(features, w1, w2, dw1, db1, dw2, db2, dw3, db3, dw4, db4, dw5, db5,
           src_nodes0, dstsrc2src0_1, dstsrc2src0_2, dstsrc2dst0_1,
           dstsrc2dst0_2, dif_mat0_1, dif_mat0_2,
           src_nodes1, dstsrc2src1_1, dstsrc2src1_2, dstsrc2dst1_1,
           dstsrc2dst1_2, dif_mat1_1, dif_mat1_2):
    src2_0, dst2_0, src2_1, dst2_1 = _sc_feature_gather(
        features, src_nodes0, dstsrc2src0_2, dstsrc2dst0_2,
        src_nodes1, dstsrc2src1_2, dstsrc2dst1_2)

    h2_0, h2_1 = _tc_layer2(dif_mat0_2, dif_mat1_2, src2_0, src2_1,
                            dst2_0, dst2_1, w1)

    s1_0, d1_0, s1_1, d1_1 = _sc_h2_gather(
        h2_0, h2_1, dstsrc2src0_1, dstsrc2dst0_1,
        dstsrc2src1_1, dstsrc2dst1_1)

    return _tc_head(s1_0, d1_0, s1_1, d1_1, dif_mat0_1, dif_mat1_1, w2,
                    dw1, db1, dw2, db2, dw3, db3, dw4, db4, dw5, db5)


# speedup vs baseline: 2.4476x; 1.1031x over previous
import functools

import jax
import jax.numpy as jnp
from jax import lax
from jax.experimental import pallas as pl
from jax.experimental.pallas import tpu as pltpu
from jax.experimental.pallas import tpu_sc as plsc

N_NODES = 100000
D = 128
B = 256
N1 = 2048
N2 = 8192

NC = 2
NS = 16
NW = NC * NS
L = 16

CHUNK = 128


def _sc_feature_gather(features, nodes0, s2_0, d2_0, nodes1, s2_1, d2_1):
    mesh = plsc.VectorSubcoreMesh(core_axis_name="c", subcore_axis_name="s")
    fdt = jax.ShapeDtypeStruct
    CS, CD = N2 // NW, N1 // NW
    TOT = 2 * (CS + CD)
    sections = ((0, 0, CS, 0), (1, CS, CD, 0),
                (2, CS + CD, CS, 1), (3, 2 * CS + CD, CD, 1))

    def body(feat, n0, sA0, dA0, n1, sA1, dA1,
             src0, dst0, src1, dst1,
             nodes_v0, nodes_v1, sel_v, idx_v, rows_v, sem, osem):
        wid = lax.axis_index("s") * NC + lax.axis_index("c")
        sels = (sA0, dA0, sA1, dA1)
        outs = (src0, dst0, src1, dst1)
        nodes = (nodes_v0, nodes_v1)
        pend = [pltpu.async_copy(n0, nodes_v0, sem),
                pltpu.async_copy(n1, nodes_v1, sem)]
        for si, off, cnt, _ in sections:
            pend.append(pltpu.async_copy(
                sels[si].at[pl.ds(wid * cnt, cnt)],
                sel_v.at[pl.ds(off, cnt)], sem))
        for h in pend:
            h.wait()
        for _, off, cnt, tbl in sections:
            for o in range(off, off + cnt, L):
                idx_v[pl.ds(o, L)] = plsc.load_gather(nodes[tbl],
                                                      [sel_v[pl.ds(o, L)]])
        pend = []
        for c in range(0, TOT, CHUNK):
            n = min(CHUNK, TOT - c)
            pend.append(pltpu.async_copy(feat.at[idx_v.at[pl.ds(c, n)]],
                                         rows_v.at[pl.ds(c, n)], sem))
        for h in pend:
            h.wait()
        pend = []
        for si, off, cnt, _ in sections:
            pend.append(pltpu.async_copy(rows_v.at[pl.ds(off, cnt)],
                                         outs[si].at[pl.ds(wid * cnt, cnt)],
                                         osem))
        for h in pend:
            h.wait()

    return pl.kernel(
        body,
        out_type=(fdt((N2, D), jnp.float32), fdt((N1, D), jnp.float32),
                  fdt((N2, D), jnp.float32), fdt((N1, D), jnp.float32)),
        mesh=mesh,
        scratch_types=(
            pltpu.VMEM((N2,), jnp.int32),
            pltpu.VMEM((N2,), jnp.int32),
            pltpu.VMEM((TOT,), jnp.int32),
            pltpu.VMEM((TOT,), jnp.int32),
            pltpu.VMEM((TOT, D), jnp.float32),
            pltpu.SemaphoreType.DMA,
            pltpu.SemaphoreType.DMA,
        ),
        compiler_params=pltpu.CompilerParams(needs_layout_passes=False),
    )(features, nodes0, s2_0, d2_0, nodes1, s2_1, d2_1)


def _sc_h2_gather(h2_0, h2_1, s1_0, d1_0, s1_1, d1_1):
    mesh = plsc.VectorSubcoreMesh(core_axis_name="c", subcore_axis_name="s")
    fdt = jax.ShapeDtypeStruct
    CS, CD = N1 // NW, B // NW
    TOT = 2 * (CS + CD)
    sections = ((0, 0, CS, 0), (1, CS, CD, 0),
                (2, CS + CD, CS, 1), (3, 2 * CS + CD, CD, 1))

    def body(t0, t1, sA0, dA0, sA1, dA1,
             src0, dst0, src1, dst1,
             idx_v, rows_v, sem, osem):
        wid = lax.axis_index("s") * NC + lax.axis_index("c")
        sels = (sA0, dA0, sA1, dA1)
        outs = (src0, dst0, src1, dst1)
        tabs = (t0, t1)
        pend = []
        for si, off, cnt, _ in sections:
            pend.append(pltpu.async_copy(
                sels[si].at[pl.ds(wid * cnt, cnt)],
                idx_v.at[pl.ds(off, cnt)], sem))
        for h in pend:
            h.wait()
        pend = []
        for si, off, cnt, tbl in sections:
            pend.append(pltpu.async_copy(
                tabs[tbl].at[idx_v.at[pl.ds(off, cnt)]],
                rows_v.at[pl.ds(off, cnt)], sem))
        for h in pend:
            h.wait()
        pend = []
        for si, off, cnt, _ in sections:
            pend.append(pltpu.async_copy(rows_v.at[pl.ds(off, cnt)],
                                         outs[si].at[pl.ds(wid * cnt, cnt)],
                                         osem))
        for h in pend:
            h.wait()

    return pl.kernel(
        body,
        out_type=(fdt((N1, D), jnp.float32), fdt((B, D), jnp.float32),
                  fdt((N1, D), jnp.float32), fdt((B, D), jnp.float32)),
        mesh=mesh,
        scratch_types=(
            pltpu.VMEM((TOT,), jnp.int32),
            pltpu.VMEM((TOT, D), jnp.float32),
            pltpu.SemaphoreType.DMA,
            pltpu.SemaphoreType.DMA,
        ),
        compiler_params=pltpu.CompilerParams(needs_layout_passes=False),
    )(h2_0, h2_1, s1_0, d1_0, s1_1, d1_1)


KBLK = 512
KB = N2 // KBLK


def _tc_layer2_body(dm0_ref, dm1_ref, src0_ref, src1_ref, dst0_ref, dst1_ref,
                    w1_ref, h20_ref, h21_ref, acc0, acc1):
    k = pl.program_id(0)

    @pl.when(k == 0)
    def _():
        acc0[...] = jnp.zeros_like(acc0)
        acc1[...] = jnp.zeros_like(acc1)

    acc0[...] += jnp.dot(dm0_ref[...], src0_ref[...],
                         preferred_element_type=jnp.float32)
    acc1[...] += jnp.dot(dm1_ref[...], src1_ref[...],
                         preferred_element_type=jnp.float32)

    @pl.when(k == KB - 1)
    def _():
        w1t = w1_ref[0:D, :]
        w1b = w1_ref[D:2 * D, :]
        h20_ref[...] = jnp.maximum(
            jnp.dot(acc0[...], w1t, preferred_element_type=jnp.float32)
            + jnp.dot(dst0_ref[...], w1b, preferred_element_type=jnp.float32),
            0.0)
        h21_ref[...] = jnp.maximum(
            jnp.dot(acc1[...], w1t, preferred_element_type=jnp.float32)
            + jnp.dot(dst1_ref[...], w1b, preferred_element_type=jnp.float32),
            0.0)


def _tc_layer2(dm0, dm1, src0, src1, dst0, dst1, w1):
    fdt = jax.ShapeDtypeStruct
    return pl.pallas_call(
        _tc_layer2_body,
        grid=(KB,),
        in_specs=[
            pl.BlockSpec((N1, KBLK), lambda k: (0, k)),
            pl.BlockSpec((N1, KBLK), lambda k: (0, k)),
            pl.BlockSpec((KBLK, D), lambda k: (k, 0)),
            pl.BlockSpec((KBLK, D), lambda k: (k, 0)),
            pl.BlockSpec((N1, D), lambda k: (0, 0)),
            pl.BlockSpec((N1, D), lambda k: (0, 0)),
            pl.BlockSpec((2 * D, D), lambda k: (0, 0)),
        ],
        out_specs=[
            pl.BlockSpec((N1, D), lambda k: (0, 0)),
            pl.BlockSpec((N1, D), lambda k: (0, 0)),
        ],
        out_shape=[fdt((N1, D), jnp.float32), fdt((N1, D), jnp.float32)],
        scratch_shapes=[pltpu.VMEM((N1, D), jnp.float32),
                        pltpu.VMEM((N1, D), jnp.float32)],
    )(dm0, dm1, src0, src1, dst0, dst1, w1)


def _tc_head_body(s10_ref, d10_ref, s11_ref, d11_ref, dmA_ref, dmB_ref,
                  w2_ref, dw1_ref, db1_ref, dw2_ref, db2_ref, dw3_ref,
                  db3_ref, dw4_ref, db4_ref, w5r_ref, db5_ref, out_ref):
    f32 = jnp.float32
    w2t = w2_ref[0:D, :]
    w2b = w2_ref[D:2 * D, :]

    agg0 = jnp.dot(dmA_ref[...], s10_ref[...], preferred_element_type=f32)
    h10 = (jnp.dot(agg0, w2t, preferred_element_type=f32)
           + jnp.dot(d10_ref[...], w2b, preferred_element_type=f32))
    agg1 = jnp.dot(dmB_ref[...], s11_ref[...], preferred_element_type=f32)
    h11 = (jnp.dot(agg1, w2t, preferred_element_type=f32)
           + jnp.dot(d11_ref[...], w2b, preferred_element_type=f32))

    ss = jnp.sum(h10 * h10 + h11 * h11, axis=1, keepdims=True)
    scale = lax.rsqrt(jnp.maximum(ss, 1e-12))

    dw1t = dw1_ref[0:D, :]
    dw1b = dw1_ref[D:2 * D, :]
    h = (jnp.dot(h10, dw1t, preferred_element_type=f32)
         + jnp.dot(h11, dw1b, preferred_element_type=f32)) * scale
    h = jnp.maximum(h + db1_ref[...], 0.0)
    h = jnp.maximum(jnp.dot(h, dw2_ref[...], preferred_element_type=f32)
                    + db2_ref[...], 0.0)
    h = jnp.maximum(jnp.dot(h, dw3_ref[...], preferred_element_type=f32)
                    + db3_ref[...], 0.0)
    h = jnp.maximum(jnp.dot(h, dw4_ref[...], preferred_element_type=f32)
                    + db4_ref[...], 0.0)
    z = jnp.sum(h * w5r_ref[...], axis=1, keepdims=True) + db5_ref[...]
    m = jnp.max(z, axis=1, keepdims=True)
    e = jnp.exp(z - m)
    out_ref[...] = e / jnp.sum(e, axis=1, keepdims=True)


def _tc_head(s10, d10, s11, d11, dmA, dmB, w2,
             dw1, db1, dw2, db2, dw3, db3, dw4, db4, dw5, db5):
    fdt = jax.ShapeDtypeStruct
    args = (s10, d10, s11, d11, dmA, dmB, w2,
            dw1, db1.reshape(1, -1), dw2, db2.reshape(1, -1),
            dw3, db3.reshape(1, -1), dw4, db4.reshape(1, -1),
            dw5.reshape(1, -1), db5.reshape(1, -1))
    return pl.pallas_call(
        _tc_head_body,
        out_shape=fdt((B, 1), jnp.float32),
    )(*args)


def kernel(features, w1, w2, dw1, db1, dw2, db2, dw3, db3, dw4, db4, dw5, db5,
           src_nodes0, dstsrc2src0_1, dstsrc2src0_2, dstsrc2dst0_1,
           dstsrc2dst0_2, dif_mat0_1, dif_mat0_2,
           src_nodes1, dstsrc2src1_1, dstsrc2src1_2, dstsrc2dst1_1,
           dstsrc2dst1_2, dif_mat1_1, dif_mat1_2):
    src2_0, dst2_0, src2_1, dst2_1 = _sc_feature_gather(
        features, src_nodes0, dstsrc2src0_2, dstsrc2dst0_2,
        src_nodes1, dstsrc2src1_2, dstsrc2dst1_2)

    h2_0, h2_1 = _tc_layer2(dif_mat0_2, dif_mat1_2, src2_0, src2_1,
                            dst2_0, dst2_1, w1)

    s1_0, d1_0, s1_1, d1_1 = _sc_h2_gather(
        h2_0, h2_1, dstsrc2src0_1, dstsrc2dst0_1,
        dstsrc2src1_1, dstsrc2dst1_1)

    return _tc_head(s1_0, d1_0, s1_1, d1_1, dif_mat0_1, dif_mat1_1, w2,
                    dw1, db1, dw2, db2, dw3, db3, dw4, db4, dw5, db5)
